# Initial kernel scaffold; baseline (speedup 1.0000x reference)
import jax
import jax.numpy as jnp
from jax.experimental import pallas as pl


def kernel(x__d, embeddings):
    raise NotImplementedError("write your pallas kernel here")


# speedup vs baseline: 1.3232x; 1.3232x over previous
import jax
import jax.numpy as jnp
from jax.experimental import pallas as pl

_K = 1024
_D = 256
_BN = 1024


def _vq_block_kernel(x_ref, e_ref, xq_ref, p_ref):
    x = x_ref[...]
    e = e_ref[...]
    u_sq = jnp.sum(jnp.square(x), axis=-1, keepdims=True)
    v_sq = jnp.sum(jnp.square(e), axis=-1)[None, :]
    dot = jax.lax.dot_general(
        x, e, (((1,), (1,)), ((), ())),
        preferred_element_type=jnp.float32)
    dist = u_sq + v_sq - 2.0 * dot
    m = jnp.min(dist, axis=-1, keepdims=True)
    iota_k = jax.lax.broadcasted_iota(jnp.int32, (x.shape[0], _K), 1)
    idx = jnp.min(jnp.where(dist == m, iota_k, _K), axis=-1, keepdims=True)
    p = (iota_k == idx).astype(jnp.float32)
    p_ref[...] = p
    xq_ref[...] = jax.lax.dot_general(
        p, e, (((1,), (0,)), ((), ())),
        preferred_element_type=jnp.float32,
        precision=jax.lax.Precision.HIGHEST)


def kernel(x__d, embeddings):
    input_size = x__d.shape[:-1]
    d = x__d.shape[-1]
    x_nd = x__d.reshape(-1, d)
    n = x_nd.shape[0]
    grid = (n // _BN,)
    xq_nd, p_nk = pl.pallas_call(
        _vq_block_kernel,
        grid=grid,
        in_specs=[
            pl.BlockSpec((_BN, _D), lambda i: (i, 0)),
            pl.BlockSpec((_K, _D), lambda i: (0, 0)),
        ],
        out_specs=[
            pl.BlockSpec((_BN, _D), lambda i: (i, 0)),
            pl.BlockSpec((_BN, _K), lambda i: (i, 0)),
        ],
        out_shape=[
            jax.ShapeDtypeStruct((n, _D), jnp.float32),
            jax.ShapeDtypeStruct((n, _K), jnp.float32),
        ],
    )(x_nd, embeddings)
    xq__d = xq_nd.reshape(input_size + (d,))
    p__k = p_nk.reshape(input_size + (_K,))
    return (xq__d, p__k)


# speedup vs baseline: 2.0108x; 1.5196x over previous
import functools

import jax
import jax.numpy as jnp
from jax import lax
from jax.experimental import pallas as pl
from jax.experimental.pallas import tpu as pltpu
from jax.experimental.pallas import tpu_sc as plsc

_K = 1024
_D = 256
_BN = 1024
_NW = 32
_CHUNK = 128


def _vq_assign_kernel(x_ref, e_ref, p_ref, idx_ref):
    x = x_ref[...]
    e = e_ref[...]
    u_sq = jnp.sum(jnp.square(x), axis=-1, keepdims=True)
    v_sq = jnp.sum(jnp.square(e), axis=-1)[None, :]
    dot = jax.lax.dot_general(
        x, e, (((1,), (1,)), ((), ())),
        preferred_element_type=jnp.float32)
    dist = u_sq + v_sq - 2.0 * dot
    m = jnp.min(dist, axis=-1, keepdims=True)
    iota_k = jax.lax.broadcasted_iota(jnp.int32, (x.shape[0], _K), 1)
    idx = jnp.min(jnp.where(dist == m, iota_k, _K), axis=-1, keepdims=True)
    p_ref[...] = (iota_k == idx).astype(jnp.float32)
    idx_ref[...] = idx


def _sc_gather(embeddings, idx_w):
    n_chunks = idx_w.shape[1]
    b_per_w = n_chunks * _CHUNK
    n = _NW * b_per_w
    mesh = plsc.VectorSubcoreMesh(core_axis_name="c", subcore_axis_name="s")

    @functools.partial(
        pl.kernel, mesh=mesh,
        out_type=jax.ShapeDtypeStruct((n, _D), jnp.float32),
        scratch_types=[
            pltpu.VMEM((n_chunks, _CHUNK), jnp.int32),
            pltpu.VMEM((_CHUNK, _D), jnp.float32),
            pltpu.VMEM((_CHUNK, _D), jnp.float32),
            pltpu.SemaphoreType.DMA,
            pltpu.SemaphoreType.DMA,
        ],
    )
    def gather_kernel(table_hbm, idx_hbm, out_hbm, idx_v, rows0, rows1, sem0, sem1):
        wid = lax.axis_index("s") * 2 + lax.axis_index("c")
        base = wid * b_per_w
        pltpu.sync_copy(idx_hbm.at[wid], idx_v)
        bufs = (rows0, rows1)
        sems = (sem0, sem1)
        copies = [pltpu.async_copy(table_hbm.at[idx_v.at[0]], rows0, sem0)]
        for j in range(n_chunks):
            if j + 1 < n_chunks:
                copies.append(pltpu.async_copy(
                    table_hbm.at[idx_v.at[j + 1]], bufs[(j + 1) % 2], sems[(j + 1) % 2]))
            copies[j].wait()
            pltpu.sync_copy(bufs[j % 2], out_hbm.at[pl.ds(base + j * _CHUNK, _CHUNK)])

    return gather_kernel(embeddings, idx_w)


def kernel(x__d, embeddings):
    input_size = x__d.shape[:-1]
    d = x__d.shape[-1]
    x_nd = x__d.reshape(-1, d)
    n = x_nd.shape[0]
    grid = (n // _BN,)
    p_nk, idx_n1 = pl.pallas_call(
        _vq_assign_kernel,
        grid=grid,
        in_specs=[
            pl.BlockSpec((_BN, _D), lambda i: (i, 0)),
            pl.BlockSpec((_K, _D), lambda i: (0, 0)),
        ],
        out_specs=[
            pl.BlockSpec((_BN, _K), lambda i: (i, 0)),
            pl.BlockSpec((_BN, 1), lambda i: (i, 0)),
        ],
        out_shape=[
            jax.ShapeDtypeStruct((n, _K), jnp.float32),
            jax.ShapeDtypeStruct((n, 1), jnp.int32),
        ],
    )(x_nd, embeddings)
    idx_w = idx_n1.reshape(_NW, n // (_NW * _CHUNK), _CHUNK)
    xq_nd = _sc_gather(embeddings, idx_w)
    xq__d = xq_nd.reshape(input_size + (d,))
    p__k = p_nk.reshape(input_size + (_K,))
    return (xq__d, p__k)
